# Initial kernel scaffold; baseline (speedup 1.0000x reference)
import jax
import jax.numpy as jnp
from jax.experimental import pallas as pl


def kernel(x, graph_idx, W1, b1, W2, b2, W3, b3, W4, b4):
    raise NotImplementedError("write your pallas kernel here")


# speedup vs baseline: 1.7685x; 1.7685x over previous
import functools

import jax
import jax.numpy as jnp
from jax import lax
from jax.experimental import pallas as pl
from jax.experimental.pallas import tpu as pltpu
from jax.experimental.pallas import tpu_sc as plsc

NUM_GRAPHS = 1024
N = 10000
D = 512
L = 16
NC = 2
NS = 16
NW = NC * NS
GPW = NUM_GRAPHS // NW
RB = 40
SB = 40
NUM_SB = N // SB
NCHUNK = D // L

_mesh = plsc.VectorSubcoreMesh(
    core_axis_name="c", subcore_axis_name="s", num_cores=NC, num_subcores=NS)


@functools.partial(
    pl.kernel,
    out_type=jax.ShapeDtypeStruct((NUM_GRAPHS, D), jnp.float32),
    mesh=_mesh,
    scratch_types=[
        pltpu.VMEM((N + L,), jnp.int32),
        pltpu.VMEM((GPW, D), jnp.float32),
        pltpu.VMEM((RB, D), jnp.float32),
    ],
)
def _segsum(x_hbm, idx_hbm, out_hbm, idx_v, acc_v, rows_v):
  wid = lax.axis_index("s") * NC + lax.axis_index("c")
  g_lo = wid * GPW
  g_hi = g_lo + GPW
  pltpu.sync_copy(idx_hbm, idx_v.at[pl.ds(0, N)])
  idx_v[pl.ds(N, L)] = jnp.full((L,), NUM_GRAPHS, jnp.int32)

  def lower_bound(target):
    def bs_body(_, carry):
      lo, hi = carry
      mid = (lo + hi) // 2
      v = idx_v[pl.ds(mid, L)][0]
      pred = (v < target) & (lo < hi)
      return jnp.where(pred, mid + 1, lo), jnp.where(pred, hi, mid)

    lo, _ = lax.fori_loop(0, 14, bs_body, (jnp.int32(0), jnp.int32(N)))
    return lo

  start = lower_bound(g_lo)
  cnt = lower_bound(g_hi) - start

  def zero_body(r, carry):
    for j in range(NCHUNK):
      acc_v[r, pl.ds(j * L, L)] = jnp.zeros((L,), jnp.float32)
    return carry

  lax.fori_loop(0, GPW, zero_body, 0)

  end = start + cnt
  blk0 = start // RB

  def batch_body(b, carry):
    base = pl.multiple_of((blk0 + b) * RB, RB)
    pltpu.sync_copy(x_hbm.at[pl.ds(base, RB)], rows_v)
    r_lo = jnp.maximum(start - base, 0)
    r_hi = jnp.minimum(end - base, RB)

    def row_body(r, rc):
      lg = idx_v[pl.ds(base + r, L)][0] - g_lo
      for j in range(NCHUNK):
        sl = pl.ds(j * L, L)
        acc_v[lg, sl] = acc_v[lg, sl] + rows_v[r, sl]
      return rc

    lax.fori_loop(r_lo, r_hi, row_body, 0)
    return carry

  lax.fori_loop(0, (end + RB - 1) // RB - blk0, batch_body, 0)
  pltpu.sync_copy(acc_v, out_hbm.at[pl.ds(g_lo, GPW)])


def _mlp_body(vn_ref, w1, b1, w2, b2, w3, b3, w4, b4, out_ref):
  vn = vn_ref[...]
  h = jnp.maximum(
      jnp.dot(vn, w1[...], preferred_element_type=jnp.float32) + b1[...], 0.0)
  h = jnp.maximum(
      jnp.dot(h, w2[...], preferred_element_type=jnp.float32) + b2[...], 0.0)
  h = jnp.maximum(
      jnp.dot(h, w3[...], preferred_element_type=jnp.float32) + b3[...], 0.0)
  out_ref[...] = (
      jnp.dot(h, w4[...], preferred_element_type=jnp.float32) + b4[...])


_mlp = pl.pallas_call(
    _mlp_body,
    out_shape=jax.ShapeDtypeStruct((NUM_GRAPHS, D), jnp.float32),
)


@functools.partial(
    pl.kernel,
    out_type=jax.ShapeDtypeStruct((N, D), jnp.float32),
    mesh=_mesh,
    scratch_types=[
        pltpu.VMEM((SB,), jnp.int32),
        pltpu.VMEM((SB, D), jnp.float32),
        pltpu.VMEM((SB, D), jnp.float32),
        pltpu.SemaphoreType.DMA,
    ],
)
def _gather_add(x_hbm, idx_hbm, h_hbm, out_hbm, idx_v, hrows_v, xrows_v, sem):
  wid = lax.axis_index("s") * NC + lax.axis_index("c")

  def body(i, carry):
    base = (wid + i * NW) * SB
    pltpu.sync_copy(idx_hbm.at[pl.ds(base, SB)], idx_v)
    cp = pltpu.async_copy(h_hbm.at[idx_v], hrows_v, sem)
    pltpu.sync_copy(x_hbm.at[pl.ds(base, SB)], xrows_v)
    cp.wait()

    def row(r, rc):
      for j in range(NCHUNK):
        sl = pl.ds(j * L, L)
        xrows_v[r, sl] = xrows_v[r, sl] + hrows_v[r, sl]
      return rc

    lax.fori_loop(0, SB, row, 0)
    pltpu.sync_copy(xrows_v, out_hbm.at[pl.ds(base, SB)])
    return carry

  lax.fori_loop(0, (NUM_SB - wid + NW - 1) // NW, body, 0)


def kernel(x, graph_idx, W1, b1, W2, b2, W3, b3, W4, b4):
  idx = graph_idx.astype(jnp.int32)
  vn = _segsum(x, idx)
  vn = lax.optimization_barrier(vn)
  h = _mlp(vn, W1, b1.reshape(1, D), W2, b2.reshape(1, D),
           W3, b3.reshape(1, D), W4, b4.reshape(1, D))
  h = lax.optimization_barrier(h)
  return _gather_add(x, idx, h)


# speedup vs baseline: 1.9067x; 1.0781x over previous
import functools

import jax
import jax.numpy as jnp
from jax import lax
from jax.experimental import pallas as pl
from jax.experimental.pallas import tpu as pltpu
from jax.experimental.pallas import tpu_sc as plsc

NUM_GRAPHS = 1024
N = 10000
D = 512
L = 16
NC = 2
NS = 16
NW = NC * NS
GPW = NUM_GRAPHS // NW
RB = 40
SB = 40
NUM_SB = N // SB
NCHUNK = D // L

_mesh = plsc.VectorSubcoreMesh(
    core_axis_name="c", subcore_axis_name="s", num_cores=NC, num_subcores=NS)


@functools.partial(
    pl.kernel,
    out_type=jax.ShapeDtypeStruct((NUM_GRAPHS, D), jnp.float32),
    mesh=_mesh,
    scratch_types=[
        pltpu.VMEM((N + L,), jnp.int32),
        pltpu.VMEM((GPW, D), jnp.float32),
        pltpu.VMEM((RB, D), jnp.float32),
    ],
)
def _segsum(x_hbm, idx_hbm, out_hbm, idx_v, acc_v, rows_v):
  wid = lax.axis_index("s") * NC + lax.axis_index("c")
  g_lo = wid * GPW
  g_hi = g_lo + GPW
  pltpu.sync_copy(idx_hbm, idx_v.at[pl.ds(0, N)])
  idx_v[pl.ds(N, L)] = jnp.full((L,), NUM_GRAPHS, jnp.int32)

  def lower_bound(target):
    def bs_body(_, carry):
      lo, hi = carry
      mid = (lo + hi) // 2
      v = idx_v[pl.ds(mid, L)][0]
      pred = (v < target) & (lo < hi)
      return jnp.where(pred, mid + 1, lo), jnp.where(pred, hi, mid)

    lo, _ = lax.fori_loop(0, 14, bs_body, (jnp.int32(0), jnp.int32(N)))
    return lo

  start = lower_bound(g_lo)
  cnt = lower_bound(g_hi) - start

  def zero_body(r, carry):
    for j in range(NCHUNK):
      acc_v[r, pl.ds(j * L, L)] = jnp.zeros((L,), jnp.float32)
    return carry

  lax.fori_loop(0, GPW, zero_body, 0)

  end = start + cnt
  blk0 = start // RB

  def batch_body(b, carry):
    base = pl.multiple_of((blk0 + b) * RB, RB)
    pltpu.sync_copy(x_hbm.at[pl.ds(base, RB)], rows_v)
    r_lo = jnp.maximum(start - base, 0)
    r_hi = jnp.minimum(end - base, RB)

    def row_body(r, rc):
      lg = idx_v[pl.ds(base + r, L)][0] - g_lo
      for j in range(NCHUNK):
        sl = pl.ds(j * L, L)
        acc_v[lg, sl] = acc_v[lg, sl] + rows_v[r, sl]
      return rc

    lax.fori_loop(r_lo, r_hi, row_body, 0)
    return carry

  lax.fori_loop(0, (end + RB - 1) // RB - blk0, batch_body, 0)
  pltpu.sync_copy(acc_v, out_hbm.at[pl.ds(g_lo, GPW)])


def _mlp_body(vn_ref, w1, b1, w2, b2, w3, b3, w4, b4, out_ref):
  vn = vn_ref[...]
  h = jnp.maximum(
      jnp.dot(vn, w1[...], preferred_element_type=jnp.float32) + b1[...], 0.0)
  h = jnp.maximum(
      jnp.dot(h, w2[...], preferred_element_type=jnp.float32) + b2[...], 0.0)
  h = jnp.maximum(
      jnp.dot(h, w3[...], preferred_element_type=jnp.float32) + b3[...], 0.0)
  out_ref[...] = (
      jnp.dot(h, w4[...], preferred_element_type=jnp.float32) + b4[...])


_mlp = pl.pallas_call(
    _mlp_body,
    out_shape=jax.ShapeDtypeStruct((NUM_GRAPHS, D), jnp.float32),
)


MAXB = 8


@functools.partial(
    pl.kernel,
    out_type=jax.ShapeDtypeStruct((N, D), jnp.float32),
    mesh=_mesh,
    scratch_types=[
        pltpu.VMEM((MAXB * SB,), jnp.int32),
        pltpu.VMEM((2, SB, D), jnp.float32),
        pltpu.VMEM((2, SB, D), jnp.float32),
        pltpu.SemaphoreType.DMA,
        pltpu.SemaphoreType.DMA,
        pltpu.SemaphoreType.DMA,
        pltpu.SemaphoreType.DMA,
        pltpu.SemaphoreType.DMA,
        pltpu.SemaphoreType.DMA,
    ],
)
def _gather_add(x_hbm, idx_hbm, h_hbm, out_hbm, idx_v, hbuf, xbuf,
                g0, g1, xs0, xs1, o0, o1):
  wid = lax.axis_index("s") * NC + lax.axis_index("c")
  b_lo = wid * NUM_SB // NW
  nb = (wid + 1) * NUM_SB // NW - b_lo
  gsem = (g0, g1)
  xsem = (xs0, xs1)
  osem = (o0, o1)

  idx_base = pl.multiple_of(b_lo * SB, 8)
  pltpu.sync_copy(idx_hbm.at[pl.ds(idx_base, MAXB * SB)], idx_v)

  def start(j):
    slot = j % 2
    base = pl.multiple_of((b_lo + j) * SB, 8)
    gd = pltpu.async_copy(
        h_hbm.at[idx_v.at[pl.ds(j * SB, SB)]], hbuf.at[slot], gsem[slot])
    xd = pltpu.async_copy(x_hbm.at[pl.ds(base, SB)], xbuf.at[slot], xsem[slot])
    return gd, xd

  def finish(j, gd, xd):
    slot = j % 2
    base = pl.multiple_of((b_lo + j) * SB, 8)
    gd.wait()
    xd.wait()

    def row(r, rc):
      for k in range(NCHUNK):
        sl = pl.ds(k * L, L)
        xbuf[slot, r, sl] = xbuf[slot, r, sl] + hbuf[slot, r, sl]
      return rc

    lax.fori_loop(0, SB, row, 0)
    return pltpu.async_copy(xbuf.at[slot], out_hbm.at[pl.ds(base, SB)],
                            osem[slot])

  def pipe(nb_s):
    def go():
      descs = {0: start(0)}
      odescs = {}
      for j in range(1, nb_s):
        if j >= 2:
          odescs[j - 2].wait()
        descs[j] = start(j)
        odescs[j - 1] = finish(j - 1, *descs[j - 1])
      odescs[nb_s - 1] = finish(nb_s - 1, *descs[nb_s - 1])
      odescs[nb_s - 2].wait()
      odescs[nb_s - 1].wait()
    return go

  pl.when(nb == 7)(pipe(7))
  pl.when(nb == 8)(pipe(8))


def kernel(x, graph_idx, W1, b1, W2, b2, W3, b3, W4, b4):
  idx = graph_idx.astype(jnp.int32)
  vn = _segsum(x, idx)
  vn = lax.optimization_barrier(vn)
  h = _mlp(vn, W1, b1.reshape(1, D), W2, b2.reshape(1, D),
           W3, b3.reshape(1, D), W4, b4.reshape(1, D))
  h = lax.optimization_barrier(h)
  return _gather_add(x, idx, h)


# speedup vs baseline: 2.0280x; 1.0637x over previous
import functools

import jax
import jax.numpy as jnp
from jax import lax
from jax.experimental import pallas as pl
from jax.experimental.pallas import tpu as pltpu
from jax.experimental.pallas import tpu_sc as plsc

NUM_GRAPHS = 1024
N = 10000
D = 512
L = 16
NC = 2
NS = 16
NW = NC * NS
GPW = NUM_GRAPHS // NW
RB = 40
SB = 40
NUM_SB = N // SB
NCHUNK = D // L

_mesh = plsc.VectorSubcoreMesh(
    core_axis_name="c", subcore_axis_name="s", num_cores=NC, num_subcores=NS)


@functools.partial(
    pl.kernel,
    out_type=jax.ShapeDtypeStruct((NUM_GRAPHS, D), jnp.float32),
    mesh=_mesh,
    scratch_types=[
        pltpu.VMEM((N + L,), jnp.int32),
        pltpu.VMEM((GPW, D), jnp.float32),
        pltpu.VMEM((2, RB, D), jnp.float32),
        pltpu.SemaphoreType.DMA,
        pltpu.SemaphoreType.DMA,
    ],
)
def _segsum(x_hbm, idx_hbm, out_hbm, idx_v, acc_v, rows_v, s0, s1):
  wid = lax.axis_index("s") * NC + lax.axis_index("c")
  g_lo = wid * GPW
  g_hi = g_lo + GPW
  pltpu.sync_copy(idx_hbm, idx_v.at[pl.ds(0, N)])
  idx_v[pl.ds(N, L)] = jnp.full((L,), NUM_GRAPHS, jnp.int32)

  def lower_bound(target):
    def bs_body(_, carry):
      lo, hi = carry
      mid = (lo + hi) // 2
      v = idx_v[pl.ds(mid, L)][0]
      pred = (v < target) & (lo < hi)
      return jnp.where(pred, mid + 1, lo), jnp.where(pred, hi, mid)

    lo, _ = lax.fori_loop(0, 14, bs_body, (jnp.int32(0), jnp.int32(N)))
    return lo

  start = lower_bound(g_lo)
  cnt = lower_bound(g_hi) - start

  def zero_body(r, carry):
    for j in range(NCHUNK):
      acc_v[r, pl.ds(j * L, L)] = jnp.zeros((L,), jnp.float32)
    return carry

  lax.fori_loop(0, GPW, zero_body, 0)

  end = start + cnt
  blk0 = start // RB
  nblk = (end + RB - 1) // RB - blk0
  sem = (s0, s1)

  def start_blk(i, slot):
    base = pl.multiple_of((blk0 + i) * RB, 8)
    pltpu.async_copy(x_hbm.at[pl.ds(base, RB)], rows_v.at[slot], sem[slot])

  def wait_blk(slot):
    pltpu.make_async_copy(
        x_hbm.at[pl.ds(0, RB)], rows_v.at[slot], sem[slot]).wait()

  pl.when(nblk > 0)(lambda: start_blk(0, 0))
  pl.when(nblk > 1)(lambda: start_blk(1, 1))

  def outer(g, carry):
    for b in range(2):
      i = g * 2 + b

      @pl.when(i < nblk)
      def _(i=i, b=b):
        wait_blk(b)
        base = (blk0 + i) * RB
        r_lo = jnp.maximum(start - base, 0)
        r_hi = jnp.minimum(end - base, RB)

        def row_body(r, rc):
          lg = idx_v[pl.ds(base + r, L)][0] - g_lo
          for j in range(NCHUNK):
            sl = pl.ds(j * L, L)
            acc_v[lg, sl] = acc_v[lg, sl] + rows_v[b, r, sl]
          return rc

        lax.fori_loop(r_lo, r_hi, row_body, 0)
        pl.when(i + 2 < nblk)(lambda: start_blk(i + 2, b))

    return carry

  lax.fori_loop(0, (nblk + 1) // 2, outer, 0)
  pltpu.sync_copy(acc_v, out_hbm.at[pl.ds(g_lo, GPW)])


def _mlp_body(vn_ref, w1, b1, w2, b2, w3, b3, w4, b4, out_ref):
  vn = vn_ref[...]
  h = jnp.maximum(
      jnp.dot(vn, w1[...], preferred_element_type=jnp.float32) + b1[...], 0.0)
  h = jnp.maximum(
      jnp.dot(h, w2[...], preferred_element_type=jnp.float32) + b2[...], 0.0)
  h = jnp.maximum(
      jnp.dot(h, w3[...], preferred_element_type=jnp.float32) + b3[...], 0.0)
  out_ref[...] = (
      jnp.dot(h, w4[...], preferred_element_type=jnp.float32) + b4[...])


_mlp = pl.pallas_call(
    _mlp_body,
    out_shape=jax.ShapeDtypeStruct((NUM_GRAPHS, D), jnp.float32),
)


MAXB = 8


@functools.partial(
    pl.kernel,
    out_type=jax.ShapeDtypeStruct((N, D), jnp.float32),
    mesh=_mesh,
    scratch_types=[
        pltpu.VMEM((MAXB * SB,), jnp.int32),
        pltpu.VMEM((2, SB, D), jnp.float32),
        pltpu.VMEM((2, SB, D), jnp.float32),
        pltpu.SemaphoreType.DMA,
        pltpu.SemaphoreType.DMA,
        pltpu.SemaphoreType.DMA,
        pltpu.SemaphoreType.DMA,
        pltpu.SemaphoreType.DMA,
        pltpu.SemaphoreType.DMA,
    ],
)
def _gather_add(x_hbm, idx_hbm, h_hbm, out_hbm, idx_v, hbuf, xbuf,
                g0, g1, xs0, xs1, o0, o1):
  wid = lax.axis_index("s") * NC + lax.axis_index("c")
  b_lo = wid * NUM_SB // NW
  nb = (wid + 1) * NUM_SB // NW - b_lo
  gsem = (g0, g1)
  xsem = (xs0, xs1)
  osem = (o0, o1)

  idx_base = pl.multiple_of(b_lo * SB, 8)
  pltpu.sync_copy(idx_hbm.at[pl.ds(idx_base, MAXB * SB)], idx_v)

  def start(j):
    slot = j % 2
    base = pl.multiple_of((b_lo + j) * SB, 8)
    gd = pltpu.async_copy(
        h_hbm.at[idx_v.at[pl.ds(j * SB, SB)]], hbuf.at[slot], gsem[slot])
    xd = pltpu.async_copy(x_hbm.at[pl.ds(base, SB)], xbuf.at[slot], xsem[slot])
    return gd, xd

  def finish(j, gd, xd):
    slot = j % 2
    base = pl.multiple_of((b_lo + j) * SB, 8)
    gd.wait()
    xd.wait()

    def row(r, rc):
      for k in range(NCHUNK):
        sl = pl.ds(k * L, L)
        xbuf[slot, r, sl] = xbuf[slot, r, sl] + hbuf[slot, r, sl]
      return rc

    lax.fori_loop(0, SB, row, 0)
    return pltpu.async_copy(xbuf.at[slot], out_hbm.at[pl.ds(base, SB)],
                            osem[slot])

  def pipe(nb_s):
    def go():
      descs = {0: start(0)}
      odescs = {}
      for j in range(1, nb_s):
        if j >= 2:
          odescs[j - 2].wait()
        descs[j] = start(j)
        odescs[j - 1] = finish(j - 1, *descs[j - 1])
      odescs[nb_s - 1] = finish(nb_s - 1, *descs[nb_s - 1])
      odescs[nb_s - 2].wait()
      odescs[nb_s - 1].wait()
    return go

  pl.when(nb == 7)(pipe(7))
  pl.when(nb == 8)(pipe(8))


def kernel(x, graph_idx, W1, b1, W2, b2, W3, b3, W4, b4):
  idx = graph_idx.astype(jnp.int32)
  vn = _segsum(x, idx)
  vn = lax.optimization_barrier(vn)
  h = _mlp(vn, W1, b1.reshape(1, D), W2, b2.reshape(1, D),
           W3, b3.reshape(1, D), W4, b4.reshape(1, D))
  h = lax.optimization_barrier(h)
  return _gather_add(x, idx, h)


# speedup vs baseline: 2.2498x; 1.1093x over previous
import functools

import jax
import jax.numpy as jnp
from jax import lax
from jax.experimental import pallas as pl
from jax.experimental.pallas import tpu as pltpu
from jax.experimental.pallas import tpu_sc as plsc

NUM_GRAPHS = 1024
N = 10000
D = 512
L = 16
NC = 2
NS = 16
NW = NC * NS
GPW = NUM_GRAPHS // NW
RB = 40
SB = 40
NUM_SB = N // SB
NCHUNK = D // L

_mesh = plsc.VectorSubcoreMesh(
    core_axis_name="c", subcore_axis_name="s", num_cores=NC, num_subcores=NS)


@functools.partial(
    pl.kernel,
    out_type=jax.ShapeDtypeStruct((NUM_GRAPHS, D), jnp.float32),
    mesh=_mesh,
    scratch_types=[
        pltpu.VMEM((N + L,), jnp.int32),
        pltpu.VMEM((GPW, D), jnp.float32),
        pltpu.VMEM((2, RB, D), jnp.float32),
        pltpu.SemaphoreType.DMA,
        pltpu.SemaphoreType.DMA,
    ],
)
def _segsum(x_hbm, idx_hbm, out_hbm, idx_v, acc_v, rows_v, s0, s1):
  wid = lax.axis_index("s") * NC + lax.axis_index("c")
  g_lo = wid * GPW
  g_hi = g_lo + GPW
  pltpu.sync_copy(idx_hbm, idx_v.at[pl.ds(0, N)])
  idx_v[pl.ds(N, L)] = jnp.full((L,), NUM_GRAPHS, jnp.int32)

  def lower_bound(target):
    def bs_body(_, carry):
      lo, hi = carry
      mid = (lo + hi) // 2
      v = idx_v[pl.ds(mid, L)][0]
      pred = (v < target) & (lo < hi)
      return jnp.where(pred, mid + 1, lo), jnp.where(pred, hi, mid)

    lo, _ = lax.fori_loop(0, 14, bs_body, (jnp.int32(0), jnp.int32(N)))
    return lo

  start = lower_bound(g_lo)
  cnt = lower_bound(g_hi) - start

  def zero_body(r, carry):
    for j in range(NCHUNK):
      acc_v[r, pl.ds(j * L, L)] = jnp.zeros((L,), jnp.float32)
    return carry

  lax.fori_loop(0, GPW, zero_body, 0)

  end = start + cnt
  blk0 = start // RB
  nblk = (end + RB - 1) // RB - blk0
  sem = (s0, s1)

  def start_blk(i, slot):
    base = pl.multiple_of((blk0 + i) * RB, 8)
    pltpu.async_copy(x_hbm.at[pl.ds(base, RB)], rows_v.at[slot], sem[slot])

  def wait_blk(slot):
    pltpu.make_async_copy(
        x_hbm.at[pl.ds(0, RB)], rows_v.at[slot], sem[slot]).wait()

  pl.when(nblk > 0)(lambda: start_blk(0, 0))
  pl.when(nblk > 1)(lambda: start_blk(1, 1))

  def outer(g, carry):
    for b in range(2):
      i = g * 2 + b

      @pl.when(i < nblk)
      def _(i=i, b=b):
        wait_blk(b)
        base = (blk0 + i) * RB
        r_lo = jnp.maximum(start - base, 0)
        r_hi = jnp.minimum(end - base, RB)

        def row_body(r, rc):
          lg = idx_v[pl.ds(base + r, L)][0] - g_lo
          for j in range(NCHUNK):
            sl = pl.ds(j * L, L)
            plsc.addupdate(acc_v.at[lg, sl], rows_v[b, r, sl])
          return rc

        lax.fori_loop(r_lo, r_hi, row_body, 0)
        pl.when(i + 2 < nblk)(lambda: start_blk(i + 2, b))

    return carry

  lax.fori_loop(0, (nblk + 1) // 2, outer, 0)
  pltpu.sync_copy(acc_v, out_hbm.at[pl.ds(g_lo, GPW)])


def _mlp_body(vn_ref, w1, b1, w2, b2, w3, b3, w4, b4, out_ref):
  vn = vn_ref[...]
  h = jnp.maximum(
      jnp.dot(vn, w1[...], preferred_element_type=jnp.float32) + b1[...], 0.0)
  h = jnp.maximum(
      jnp.dot(h, w2[...], preferred_element_type=jnp.float32) + b2[...], 0.0)
  h = jnp.maximum(
      jnp.dot(h, w3[...], preferred_element_type=jnp.float32) + b3[...], 0.0)
  out_ref[...] = (
      jnp.dot(h, w4[...], preferred_element_type=jnp.float32) + b4[...])


_mlp = pl.pallas_call(
    _mlp_body,
    out_shape=jax.ShapeDtypeStruct((NUM_GRAPHS, D), jnp.float32),
)


MAXB = 8


@functools.partial(
    pl.kernel,
    out_type=jax.ShapeDtypeStruct((N, D), jnp.float32),
    mesh=_mesh,
    scratch_types=[
        pltpu.VMEM((MAXB * SB,), jnp.int32),
        pltpu.VMEM((2, SB, D), jnp.float32),
        pltpu.VMEM((2, SB, D), jnp.float32),
        pltpu.SemaphoreType.DMA,
        pltpu.SemaphoreType.DMA,
        pltpu.SemaphoreType.DMA,
        pltpu.SemaphoreType.DMA,
        pltpu.SemaphoreType.DMA,
        pltpu.SemaphoreType.DMA,
    ],
)
def _gather_add(x_hbm, idx_hbm, h_hbm, out_hbm, idx_v, hbuf, xbuf,
                g0, g1, xs0, xs1, o0, o1):
  wid = lax.axis_index("s") * NC + lax.axis_index("c")
  b_lo = wid * NUM_SB // NW
  nb = (wid + 1) * NUM_SB // NW - b_lo
  gsem = (g0, g1)
  xsem = (xs0, xs1)
  osem = (o0, o1)

  idx_base = pl.multiple_of(b_lo * SB, 8)
  pltpu.sync_copy(idx_hbm.at[pl.ds(idx_base, MAXB * SB)], idx_v)

  def start(j):
    slot = j % 2
    base = pl.multiple_of((b_lo + j) * SB, 8)
    gd = pltpu.async_copy(
        h_hbm.at[idx_v.at[pl.ds(j * SB, SB)]], hbuf.at[slot], gsem[slot])
    xd = pltpu.async_copy(x_hbm.at[pl.ds(base, SB)], xbuf.at[slot], xsem[slot])
    return gd, xd

  def finish(j, gd, xd):
    slot = j % 2
    base = pl.multiple_of((b_lo + j) * SB, 8)
    gd.wait()
    xd.wait()

    def row(r, rc):
      for k in range(NCHUNK):
        sl = pl.ds(k * L, L)
        plsc.addupdate(xbuf.at[slot, r, sl], hbuf[slot, r, sl])
      return rc

    lax.fori_loop(0, SB, row, 0)
    return pltpu.async_copy(xbuf.at[slot], out_hbm.at[pl.ds(base, SB)],
                            osem[slot])

  def pipe(nb_s):
    def go():
      descs = {0: start(0)}
      odescs = {}
      for j in range(1, nb_s):
        if j >= 2:
          odescs[j - 2].wait()
        descs[j] = start(j)
        odescs[j - 1] = finish(j - 1, *descs[j - 1])
      odescs[nb_s - 1] = finish(nb_s - 1, *descs[nb_s - 1])
      odescs[nb_s - 2].wait()
      odescs[nb_s - 1].wait()
    return go

  pl.when(nb == 7)(pipe(7))
  pl.when(nb == 8)(pipe(8))


def kernel(x, graph_idx, W1, b1, W2, b2, W3, b3, W4, b4):
  idx = graph_idx.astype(jnp.int32)
  vn = _segsum(x, idx)
  vn = lax.optimization_barrier(vn)
  h = _mlp(vn, W1, b1.reshape(1, D), W2, b2.reshape(1, D),
           W3, b3.reshape(1, D), W4, b4.reshape(1, D))
  h = lax.optimization_barrier(h)
  return _gather_add(x, idx, h)
